# baseline (speedup 1.0000x reference)
import functools

import jax
import jax.numpy as jnp
from jax import lax
from jax.experimental import pallas as pl
from jax.experimental.pallas import tpu as pltpu
from jax.experimental.pallas import tpu_sc as plsc

BATCH = 16384
EMBED_DIM = 64

_NUM_CORES = 2
_NUM_SUBCORES = 16
_NUM_WORKERS = _NUM_CORES * _NUM_SUBCORES
_B_PER_W = BATCH // _NUM_WORKERS
_CHUNK = 128
_NUM_CHUNKS = _B_PER_W // _CHUNK


def _gather_body(user_hbm, item_hbm, ut_hbm, it_hbm, ue_out, ie_out,
                 uidx_v, iidx_v, urows_v, irows_v, sem):
    wid = lax.axis_index("s") * _NUM_CORES + lax.axis_index("c")
    base = wid * _B_PER_W
    pltpu.sync_copy(user_hbm.at[pl.ds(wid * _NUM_CHUNKS, _NUM_CHUNKS)], uidx_v)
    pltpu.sync_copy(item_hbm.at[pl.ds(wid * _NUM_CHUNKS, _NUM_CHUNKS)], iidx_v)
    copies = []
    for j in range(_NUM_CHUNKS):
        copies.append(pltpu.async_copy(
            ut_hbm.at[uidx_v.at[j]],
            urows_v.at[pl.ds(j * _CHUNK, _CHUNK)],
            sem,
        ))
        copies.append(pltpu.async_copy(
            it_hbm.at[iidx_v.at[j]],
            irows_v.at[pl.ds(j * _CHUNK, _CHUNK)],
            sem,
        ))
    for c in copies:
        c.wait()
    pltpu.sync_copy(urows_v, ue_out.at[pl.ds(base, _B_PER_W)])
    pltpu.sync_copy(irows_v, ie_out.at[pl.ds(base, _B_PER_W)])


def kernel(user, item, user_table, item_table):
    mesh = plsc.VectorSubcoreMesh(core_axis_name="c", subcore_axis_name="s")
    out_type = (
        jax.ShapeDtypeStruct((BATCH, EMBED_DIM), jnp.float32),
        jax.ShapeDtypeStruct((BATCH, EMBED_DIM), jnp.float32),
    )
    k = functools.partial(
        pl.kernel,
        mesh=mesh,
        out_type=out_type,
        scratch_types=[
            pltpu.VMEM((_NUM_CHUNKS, _CHUNK), jnp.int32),
            pltpu.VMEM((_NUM_CHUNKS, _CHUNK), jnp.int32),
            pltpu.VMEM((_B_PER_W, EMBED_DIM), jnp.float32),
            pltpu.VMEM((_B_PER_W, EMBED_DIM), jnp.float32),
            pltpu.SemaphoreType.DMA,
        ],
        compiler_params=pltpu.CompilerParams(use_tc_tiling_on_sc=False),
    )(_gather_body)
    user2d = user.reshape(BATCH // _CHUNK, _CHUNK)
    item2d = item.reshape(BATCH // _CHUNK, _CHUNK)
    return k(user2d, item2d, user_table, item_table)
